# baseline (speedup 1.0000x reference)
import functools

import numpy as np
import jax
import jax.numpy as jnp
from jax import lax
from jax.experimental import pallas as pl
from jax.experimental.pallas import tpu as pltpu
from jax.experimental.pallas import tpu_sc as plsc

_H = 224
_K = 17
_S = 4
_NW = 52
_L = 5
_G_MEAN = 85.384
_G_STD = 53.798
_THRESH = (0.5, _G_MEAN - _G_STD, _G_MEAN, _G_MEAN + _G_STD)

_OFFSETS = [(0, 1), (1, 1), (1, 0), (1, -1),
            (0, 2), (1, 1), (2, 0), (1, -1),
            (0, 4), (3, 3), (4, 0), (3, -3)]
_UNIQUE = []
for _o in _OFFSETS:
    if _o not in _UNIQUE:
        _UNIQUE.append(_o)
_T2U = [_UNIQUE.index(_o) for _o in _OFFSETS]
_NU = len(_UNIQUE)
_NBIN = 32
_HSLAB = _NU * _NBIN * _NW


def _build_mats():
    r = np.arange(_H)[None, :]
    k4 = (_S * np.arange(_NW))[:, None]
    m17 = ((r >= k4) & (r < k4 + _K)).astype(np.float32)
    sel = np.zeros((_NW, _H), np.float32)
    sel[np.arange(_NW), _S * np.arange(_NW)] = 1.0
    return np.stack([m17, sel])


def _dot(a, b):
    return lax.dot_general(a, b, (((1,), (0,)), ((), ())),
                           precision=lax.Precision.HIGHEST,
                           preferred_element_type=jnp.float32)


def _dot_nt(a, b):
    return lax.dot_general(a, b, (((1,), (1,)), ((), ())),
                           precision=lax.Precision.HIGHEST,
                           preferred_element_type=jnp.float32)


def _shift(m, dr, dc, fill):
    src = m[dr:, max(dc, 0):_H + min(dc, 0)]
    rows = _H - dr
    if dc > 0:
        src = jnp.concatenate(
            [src, jnp.full((rows, dc), fill, m.dtype)], axis=1)
    elif dc < 0:
        src = jnp.concatenate(
            [jnp.full((rows, -dc), fill, m.dtype), src], axis=1)
    if dr > 0:
        src = jnp.concatenate(
            [src, jnp.full((dr, _H), fill, m.dtype)], axis=0)
    return src


def _body_a(x_ref, mats_ref, feat_ref, codes_ref):
    x = x_ref[0, 0]
    m17 = mats_ref[0]
    sel = mats_ref[1][:, :_H - _K + 1]

    npx = float(_K * _K)
    s1 = _dot_nt(_dot(m17, x), m17)
    s2 = _dot_nt(_dot(m17, x * x), m17)
    mean = s1 / npx
    var = jnp.maximum(s2 / npx - mean * mean, 0.0)
    std = jnp.sqrt(var)

    def box_red(m, op):
        cm = m[:, 0:_H - _K + 1]
        for i in range(1, _K):
            cm = op(cm, m[:, i:_H - _K + 1 + i])
        csel = _dot_nt(cm, sel)
        rm = csel[0:_H - _K + 1, :]
        for i in range(1, _K):
            rm = op(rm, csel[i:_H - _K + 1 + i, :])
        return _dot(sel, rm)

    wmax = box_red(x, jnp.maximum)
    wmin = box_red(x, jnp.minimum)

    means_n = mean / _G_MEAN
    std_n = std / _G_STD
    mx_n = (wmax - means_n) / _G_STD
    mn_n = (means_n - wmin) / _G_STD

    qlev = jnp.zeros_like(x)
    for t in _THRESH:
        qlev = qlev + (x >= t).astype(jnp.float32)

    hc = [_dot_nt(_dot(m17, (qlev == float(a)).astype(jnp.float32)), m17)
          for a in range(1, _L)]
    hs = hc[0] + hc[1] + hc[2] + hc[3]
    hsg = jnp.where(hs == 0.0, 1.0, hs)

    inv_w = 1.0 / float(_NW * _NW)
    vals = [jnp.sum(means_n), jnp.sum(std_n), jnp.sum(mx_n), jnp.sum(mn_n)]
    vals += [jnp.sum(h / hsg) for h in hc]
    row = jnp.concatenate([(v * inv_w).reshape(1, 1) for v in vals], axis=1)
    feat_ref[...] = row[None]

    qi = qlev.astype(jnp.int32)
    for u, (dr, dc) in enumerate(_UNIQUE):
        qs = _shift(qlev, dr, dc, 0.0)
        code = qi * _L + qs.astype(jnp.int32)
        for k1 in range(_NW):
            codes_ref[0, k1, u] = code[_S * k1:_S * k1 + _K, :]


def _sc_body(codes_hbm, out_hbm, codes_v, hist_v):
    nb = codes_hbm.shape[0]
    ntask = nb * _NW
    niter = (ntask + 31) // 32
    cid = lax.axis_index("c")
    sid = lax.axis_index("s")
    wid = sid * 2 + cid
    lane = lax.iota(jnp.int32, 16)
    ones = jnp.ones((16,), jnp.float32)

    def task_body(it, carry):
        task = wid + 32 * it

        @pl.when(task < ntask)
        def _():
            bidx = task // _NW
            k1 = task % _NW
            pltpu.sync_copy(codes_hbm.at[bidx, k1, 0], codes_v)

            def zbody(i, c):
                hist_v[pl.ds(i * 16, 16)] = jnp.zeros((16,), jnp.float32)
                return c
            lax.fori_loop(0, _HSLAB // 16, zbody, 0)

            def k2body(k2, c):
                cols = k2 * _S + lane
                for u, (dr, dc) in enumerate(_UNIQUE):
                    rows = _K - dr
                    wdt = _K - abs(dc)
                    lo = max(0, -dc)
                    base = (u * _NBIN) * _NW + k2
                    for r in range(rows):
                        codes = plsc.load_gather(
                            codes_v, [(u * _K + r) * _H + lo + cols])
                        if wdt < 16:
                            codes = jnp.where(lane < wdt, codes, 26)
                        plsc.addupdate_scatter(
                            hist_v, [base + codes * _NW], ones)
                    if wdt == _K:
                        strip = plsc.load_gather(
                            codes_v,
                            [u * (_K * _H) + lane * _H + (cols - lane + 16)])
                        sidx = jnp.where(lane < rows, strip, 26)
                        plsc.addupdate_scatter(
                            hist_v, [base + sidx * _NW], ones)
                return c
            lax.fori_loop(0, _NW, k2body, 0)
            pltpu.sync_copy(hist_v, out_hbm.at[bidx, k1, 0])
        return carry

    lax.fori_loop(0, niter, task_body, 0)


def _glcm_stats(cnt, inv):
    n = [[(cnt[a][b] + cnt[b][a]) * inv for b in range(_L)]
         for a in range(_L)]
    z = jnp.zeros_like(n[0][0])
    con, hom, ene2, ent = z, z, z, z
    rowsum = [z for _ in range(_L)]
    colsum = [z for _ in range(_L)]
    for a in range(_L):
        for b in range(_L):
            nab = n[a][b]
            d2 = float((a - b) ** 2)
            if d2:
                con = con + d2 * nab
            hom = hom + nab * (1.0 / (1.0 + d2))
            ene2 = ene2 + nab * nab
            ent = ent - nab * jnp.log2(nab + 1e-8)
            rowsum[a] = rowsum[a] + nab
            colsum[b] = colsum[b] + nab
    mi, mj = z, z
    for a in range(1, _L):
        mi = mi + float(a) * rowsum[a]
        mj = mj + float(a) * colsum[a]
    vi, vj, cov = z, z, z
    for a in range(_L):
        da = float(a) - mi
        vi = vi + da * da * rowsum[a]
        db = float(a) - mj
        vj = vj + db * db * colsum[a]
    for a in range(_L):
        da = float(a) - mi
        for b in range(_L):
            cov = cov + da * (float(b) - mj) * n[a][b]
    stdi = jnp.sqrt(vi)
    stdj = jnp.sqrt(vj)
    den = stdi * stdj
    corr = jnp.where((stdi < 1e-15) | (stdj < 1e-15), 1.0,
                     cov / jnp.where(den == 0.0, 1.0, den))
    energy = jnp.sqrt(ene2)
    return tuple(jnp.sum(m) for m in (con, hom, energy, corr, ent))


def _body_c(cnt_ref, out_ref):
    uniq_feats = []
    for u, (dr, dc) in enumerate(_UNIQUE):
        cnt = [[cnt_ref[0, :, u, a * _L + b, :] for b in range(_L)]
               for a in range(_L)]
        inv = 1.0 / float(2 * (_K - dr) * (_K - abs(dc)))
        uniq_feats.append(_glcm_stats(cnt, inv))
    inv_w = 1.0 / float(_NW * _NW)
    vals = []
    for f in range(5):
        vals += [uniq_feats[_T2U[t]][f] for t in range(len(_OFFSETS))]
    row = jnp.concatenate([(v * inv_w).reshape(1, 1) for v in vals], axis=1)
    out_ref[...] = row[None]


def kernel(x):
    b = x.shape[0]
    mats = jnp.asarray(_build_mats())
    feats_a, codes = pl.pallas_call(
        _body_a,
        grid=(b,),
        in_specs=[
            pl.BlockSpec((1, 1, _H, _H), lambda i: (i, 0, 0, 0)),
            pl.BlockSpec(mats.shape, lambda i: (0, 0, 0)),
        ],
        out_specs=[
            pl.BlockSpec((1, 1, 8), lambda i: (i, 0, 0)),
            pl.BlockSpec((1, _NW, _NU, _K, _H), lambda i: (i, 0, 0, 0, 0)),
        ],
        out_shape=[
            jax.ShapeDtypeStruct((b, 1, 8), jnp.float32),
            jax.ShapeDtypeStruct((b, _NW, _NU, _K, _H), jnp.int32),
        ],
    )(x, mats)

    sc = functools.partial(
        pl.kernel,
        mesh=plsc.VectorSubcoreMesh(core_axis_name="c", subcore_axis_name="s"),
        compiler_params=pltpu.CompilerParams(
            use_tc_tiling_on_sc=False, needs_layout_passes=False),
        out_type=jax.ShapeDtypeStruct((b, _NW, 1, _HSLAB), jnp.float32),
        scratch_types=[
            pltpu.VMEM((_NU * _K * _H,), jnp.int32),
            pltpu.VMEM((_HSLAB,), jnp.float32),
        ],
    )(_sc_body)
    slabs = codes.reshape(b, _NW, 1, _NU * _K * _H)
    counts = sc(slabs).reshape(b, _NW, _NU, _NBIN, _NW)

    feats_c = pl.pallas_call(
        _body_c,
        grid=(b,),
        in_specs=[
            pl.BlockSpec((1, _NW, _NU, _NBIN, _NW),
                         lambda i: (i, 0, 0, 0, 0)),
        ],
        out_specs=pl.BlockSpec((1, 1, 60), lambda i: (i, 0, 0)),
        out_shape=jax.ShapeDtypeStruct((b, 1, 60), jnp.float32),
    )(counts)

    return jnp.concatenate(
        [feats_a.reshape(b, 8), feats_c.reshape(b, 60)], axis=1)


# speedup vs baseline: 3.2798x; 3.2798x over previous
import functools

import numpy as np
import jax
import jax.numpy as jnp
from jax import lax
from jax.experimental import pallas as pl

_H = 224
_K = 17
_S = 4
_NW = 52
_L = 5
_G_MEAN = 85.384
_G_STD = 53.798
_THRESH = (0.5, _G_MEAN - _G_STD, _G_MEAN, _G_MEAN + _G_STD)

_OFFSETS = [(0, 1), (1, 1), (1, 0), (1, -1),
            (0, 2), (1, 1), (2, 0), (1, -1),
            (0, 4), (3, 3), (4, 0), (3, -3)]
_UNIQUE = []
for _o in _OFFSETS:
    if _o not in _UNIQUE:
        _UNIQUE.append(_o)
_T2U = [_UNIQUE.index(_o) for _o in _OFFSETS]
_NU = len(_UNIQUE)


def _build_mats():
    r = np.arange(_H)[None, :]
    k4 = (_S * np.arange(_NW))[:, None]
    mats = []
    for dr, _dc in _UNIQUE:
        mats.append(((r >= k4) & (r < k4 + _K - dr)).astype(np.float32))
    for _dr, dc in _UNIQUE:
        lo, hi = max(0, -dc), max(0, dc)
        mats.append(((r >= k4 + lo) & (r < k4 + _K - hi)).astype(np.float32))
    mats.append(((r >= k4) & (r < k4 + _K)).astype(np.float32))
    sel = np.zeros((_NW, _H), np.float32)
    sel[np.arange(_NW), _S * np.arange(_NW)] = 1.0
    mats.append(sel)
    return np.stack(mats)


def _dot(a, b):
    return lax.dot_general(a, b, (((1,), (0,)), ((), ())),
                           precision=lax.Precision.HIGHEST,
                           preferred_element_type=jnp.float32)


def _dot_nt(a, b):
    return lax.dot_general(a, b, (((1,), (1,)), ((), ())),
                           precision=lax.Precision.HIGHEST,
                           preferred_element_type=jnp.float32)


def _shift(m, dr, dc, fill):
    src = m[dr:, max(dc, 0):_H + min(dc, 0)]
    rows = _H - dr
    if dc > 0:
        src = jnp.concatenate(
            [src, jnp.full((rows, dc), fill, m.dtype)], axis=1)
    elif dc < 0:
        src = jnp.concatenate(
            [jnp.full((rows, -dc), fill, m.dtype), src], axis=1)
    if dr > 0:
        src = jnp.concatenate(
            [src, jnp.full((dr, _H), fill, m.dtype)], axis=0)
    return src


def _body(x_ref, mats_ref, out_ref):
    x = x_ref[0, 0]
    m17 = mats_ref[2 * _NU]
    sel = mats_ref[2 * _NU + 1][:, :_H - _K + 1]

    npx = float(_K * _K)
    s1 = _dot_nt(_dot(m17, x), m17)
    s2 = _dot_nt(_dot(m17, x * x), m17)
    mean = s1 / npx
    var = jnp.maximum(s2 / npx - mean * mean, 0.0)
    std = jnp.sqrt(var)

    def box_red(m, op):
        cm = m[:, 0:_H - _K + 1]
        for i in range(1, _K):
            cm = op(cm, m[:, i:_H - _K + 1 + i])
        csel = _dot_nt(cm, sel)
        rm = csel[0:_H - _K + 1, :]
        for i in range(1, _K):
            rm = op(rm, csel[i:_H - _K + 1 + i, :])
        return _dot(sel, rm)

    wmax = box_red(x, jnp.maximum)
    wmin = box_red(x, jnp.minimum)

    means_n = mean / _G_MEAN
    std_n = std / _G_STD
    mx_n = (wmax - means_n) / _G_STD
    mn_n = (means_n - wmin) / _G_STD

    qlev = jnp.zeros_like(x)
    for t in _THRESH:
        qlev = qlev + (x >= t).astype(jnp.float32)
    onehot = [(qlev == float(a)).astype(jnp.float32) for a in range(_L)]

    hc = [_dot_nt(_dot(m17, onehot[a]), m17) for a in range(1, _L)]
    hs = hc[0] + hc[1] + hc[2] + hc[3]
    hsg = jnp.where(hs == 0.0, 1.0, hs)
    hist = [h / hsg for h in hc]

    uniq_feats = []
    for u, (dr, dc) in enumerate(_UNIQUE):
        ar = mats_ref[u]
        ac = mats_ref[_NU + u]
        qs = _shift(qlev, dr, dc, -1.0)
        sh = [(qs == float(b)).astype(jnp.float32) for b in range(_L)]
        cnt = [[_dot_nt(_dot(ar, onehot[a] * sh[b]), ac)
                for b in range(_L)] for a in range(_L)]
        inv = 1.0 / float(2 * (_K - dr) * (_K - abs(dc)))
        N = [[(cnt[a][b] + cnt[b][a]) * inv for b in range(_L)]
             for a in range(_L)]
        con = jnp.zeros_like(N[0][0])
        hom = jnp.zeros_like(con)
        ene2 = jnp.zeros_like(con)
        ent = jnp.zeros_like(con)
        mi = jnp.zeros_like(con)
        mj = jnp.zeros_like(con)
        rowsum = [jnp.zeros_like(con) for _ in range(_L)]
        colsum = [jnp.zeros_like(con) for _ in range(_L)]
        for a in range(_L):
            for b in range(_L):
                nab = N[a][b]
                d2 = float((a - b) ** 2)
                if d2:
                    con = con + d2 * nab
                hom = hom + nab * (1.0 / (1.0 + d2))
                ene2 = ene2 + nab * nab
                ent = ent - nab * (jnp.log2(nab + 1e-8))
                rowsum[a] = rowsum[a] + nab
                colsum[b] = colsum[b] + nab
        for a in range(1, _L):
            mi = mi + float(a) * rowsum[a]
            mj = mj + float(a) * colsum[a]
        vi = jnp.zeros_like(con)
        vj = jnp.zeros_like(con)
        cov = jnp.zeros_like(con)
        for a in range(_L):
            da = float(a) - mi
            vi = vi + da * da * rowsum[a]
            db = float(a) - mj
            vj = vj + db * db * colsum[a]
        for a in range(_L):
            da = float(a) - mi
            for b in range(_L):
                cov = cov + da * (float(b) - mj) * N[a][b]
        stdi = jnp.sqrt(vi)
        stdj = jnp.sqrt(vj)
        den = stdi * stdj
        corr = jnp.where((stdi < 1e-15) | (stdj < 1e-15), 1.0,
                         cov / jnp.where(den == 0.0, 1.0, den))
        energy = jnp.sqrt(ene2)
        uniq_feats.append(tuple(jnp.sum(m)
                                for m in (con, hom, energy, corr, ent)))

    inv_w = 1.0 / float(_NW * _NW)
    vals = [jnp.sum(means_n), jnp.sum(std_n), jnp.sum(mx_n), jnp.sum(mn_n)]
    vals += [jnp.sum(h) for h in hist]
    for f in range(5):
        vals += [uniq_feats[_T2U[t]][f] for t in range(len(_OFFSETS))]
    row = jnp.concatenate([(v * inv_w).reshape(1, 1) for v in vals], axis=1)
    out_ref[...] = row[None]


def kernel(x):
    b = x.shape[0]
    mats = jnp.asarray(_build_mats())
    return pl.pallas_call(
        _body,
        grid=(b,),
        in_specs=[
            pl.BlockSpec((1, 1, _H, _H), lambda i: (i, 0, 0, 0)),
            pl.BlockSpec(mats.shape, lambda i: (0, 0, 0)),
        ],
        out_specs=pl.BlockSpec((1, 1, 68), lambda i: (i, 0, 0)),
        out_shape=jax.ShapeDtypeStruct((b, 1, 68), jnp.float32),
    )(x, mats).reshape(b, 68)


# speedup vs baseline: 11.6689x; 3.5578x over previous
import functools

import numpy as np
import jax
import jax.numpy as jnp
from jax import lax
from jax.experimental import pallas as pl

_H = 224
_K = 17
_S = 4
_NW = 52
_L = 5
_G_MEAN = 85.384
_G_STD = 53.798
_THRESH = (0.5, _G_MEAN - _G_STD, _G_MEAN, _G_MEAN + _G_STD)

_OFFSETS = [(0, 1), (1, 1), (1, 0), (1, -1),
            (0, 2), (1, 1), (2, 0), (1, -1),
            (0, 4), (3, 3), (4, 0), (3, -3)]
_UNIQUE = []
for _o in _OFFSETS:
    if _o not in _UNIQUE:
        _UNIQUE.append(_o)
_T2U = [_UNIQUE.index(_o) for _o in _OFFSETS]
_NU = len(_UNIQUE)


def _build_mats():
    r = np.arange(_H)[None, :]
    k4 = (_S * np.arange(_NW))[:, None]
    mats = []
    for dr, _dc in _UNIQUE:
        mats.append(((r >= k4) & (r < k4 + _K - dr)).astype(np.float32))
    for _dr, dc in _UNIQUE:
        lo, hi = max(0, -dc), max(0, dc)
        mats.append(((r >= k4 + lo) & (r < k4 + _K - hi)).astype(np.float32))
    mats.append(((r >= k4) & (r < k4 + _K)).astype(np.float32))
    sel = np.zeros((_NW, _H), np.float32)
    sel[np.arange(_NW), _S * np.arange(_NW)] = 1.0
    mats.append(sel)
    return np.stack(mats)


def _dot_hi(a, b):
    return lax.dot_general(a, b, (((1,), (0,)), ((), ())),
                           precision=lax.Precision.HIGHEST,
                           preferred_element_type=jnp.float32)


def _dot_nt_hi(a, b):
    return lax.dot_general(a, b, (((1,), (1,)), ((), ())),
                           precision=lax.Precision.HIGHEST,
                           preferred_element_type=jnp.float32)


def _dot(a, b):
    return lax.dot_general(a, b, (((1,), (0,)), ((), ())),
                           preferred_element_type=jnp.float32)


def _dot_nt(a, b):
    return lax.dot_general(a, b, (((1,), (1,)), ((), ())),
                           preferred_element_type=jnp.float32)


def _shift(m, dr, dc, fill):
    src = m[dr:, max(dc, 0):_H + min(dc, 0)]
    rows = _H - dr
    if dc > 0:
        src = jnp.concatenate(
            [src, jnp.full((rows, dc), fill, m.dtype)], axis=1)
    elif dc < 0:
        src = jnp.concatenate(
            [jnp.full((rows, -dc), fill, m.dtype), src], axis=1)
    if dr > 0:
        src = jnp.concatenate(
            [src, jnp.full((dr, _H), fill, m.dtype)], axis=0)
    return src


def _win_reduce(m, op, axis):
    def sl(a, off, size):
        if axis == 0:
            return a[off:off + size, :]
        return a[:, off:off + size]
    d = m.shape[axis]
    w2 = op(sl(m, 0, d - 1), sl(m, 1, d - 1))
    w4 = op(sl(w2, 0, d - 3), sl(w2, 2, d - 3))
    w8 = op(sl(w4, 0, d - 7), sl(w4, 4, d - 7))
    w16 = op(sl(w8, 0, d - 15), sl(w8, 8, d - 15))
    return op(sl(w16, 0, d - 16), sl(m, 16, d - 16))


def _body(x_ref, mats_ref, out_ref):
    x = x_ref[0, 0]
    m17 = mats_ref[2 * _NU]
    sel = mats_ref[2 * _NU + 1][:, :_H - _K + 1]

    npx = float(_K * _K)
    s1 = _dot_nt_hi(_dot_hi(m17, x), m17)
    s2 = _dot_nt_hi(_dot_hi(m17, x * x), m17)
    mean = s1 / npx
    var = jnp.maximum(s2 / npx - mean * mean, 0.0)
    std = jnp.sqrt(var)

    def box_red(m, op):
        cm = _win_reduce(m, op, axis=1)
        csel = _dot_nt_hi(cm, sel)
        rm = _win_reduce(csel, op, axis=0)
        return _dot_hi(sel, rm)

    wmax = box_red(x, jnp.maximum)
    wmin = box_red(x, jnp.minimum)

    means_n = mean / _G_MEAN
    std_n = std / _G_STD
    mx_n = (wmax - means_n) / _G_STD
    mn_n = (means_n - wmin) / _G_STD

    qlev = jnp.zeros_like(x)
    for t in _THRESH:
        qlev = qlev + (x >= t).astype(jnp.float32)
    onehot = [(qlev == float(a)).astype(jnp.float32) for a in range(_L)]

    hc = [_dot_nt(_dot(m17, onehot[a]), m17) for a in range(1, _L)]
    hs = hc[0] + hc[1] + hc[2] + hc[3]
    hsg = jnp.where(hs == 0.0, 1.0, hs)
    hist = [h / hsg for h in hc]

    obig = jnp.concatenate(
        [onehot[a] for a in range(_L) for _ in range(_L)], axis=0)

    uniq_feats = []
    for u, (dr, dc) in enumerate(_UNIQUE):
        ar = mats_ref[u]
        ac = mats_ref[_NU + u]
        qs = _shift(qlev, dr, dc, -1.0)
        sh = [(qs == float(b)).astype(jnp.float32) for b in range(_L)]
        scat = jnp.concatenate(sh, axis=0)
        sbig = jnp.concatenate([scat] * _L, axis=0)
        pstack = obig * sbig
        tstack = _dot_nt(pstack, ac)
        cnt = [[_dot(ar, tstack[(_L * a + b) * _H:(_L * a + b + 1) * _H])
                for b in range(_L)] for a in range(_L)]
        inv = 1.0 / float(2 * (_K - dr) * (_K - abs(dc)))
        nst = jnp.stack([(cnt[a][b] + cnt[b][a]) * inv
                         for a in range(_L) for b in range(_L)])
        s_p = jnp.sum(jnp.sum(nst, axis=2), axis=1)
        con = jnp.float32(0.0)
        hom = jnp.float32(0.0)
        for a in range(_L):
            for bb in range(_L):
                d2 = float((a - bb) ** 2)
                spv = s_p[_L * a + bb]
                if d2:
                    con = con + d2 * spv
                hom = hom + (1.0 / (1.0 + d2)) * spv
        ent = -jnp.sum(nst * jnp.log2(nst + 1e-8))
        energy = jnp.sum(jnp.sqrt(jnp.sum(nst * nst, axis=0)))
        n4 = nst.reshape(_L, _L, _NW, _NW)
        ra = jnp.sum(n4, axis=1)
        cb = jnp.sum(n4, axis=0)
        mi = jnp.zeros_like(ra[0])
        mj = jnp.zeros_like(mi)
        for a in range(1, _L):
            mi = mi + float(a) * ra[a]
            mj = mj + float(a) * cb[a]
        vi = jnp.zeros_like(mi)
        vj = jnp.zeros_like(mi)
        for a in range(_L):
            da = float(a) - mi
            vi = vi + da * da * ra[a]
            db = float(a) - mj
            vj = vj + db * db * cb[a]
        cov = jnp.zeros_like(mi)
        for a in range(_L):
            da = float(a) - mi
            for b in range(_L):
                cov = cov + da * (float(b) - mj) * n4[a, b]
        stdi = jnp.sqrt(vi)
        stdj = jnp.sqrt(vj)
        den = stdi * stdj
        corr = jnp.where((stdi < 1e-15) | (stdj < 1e-15), 1.0,
                         cov / jnp.where(den == 0.0, 1.0, den))
        uniq_feats.append((con, hom, energy, jnp.sum(corr), ent))

    inv_w = 1.0 / float(_NW * _NW)
    vals = [jnp.sum(means_n), jnp.sum(std_n), jnp.sum(mx_n), jnp.sum(mn_n)]
    vals += [jnp.sum(h) for h in hist]
    for f in range(5):
        vals += [uniq_feats[_T2U[t]][f] for t in range(len(_OFFSETS))]
    row = jnp.concatenate([(v * inv_w).reshape(1, 1) for v in vals], axis=1)
    out_ref[...] = row[None]


def kernel(x):
    b = x.shape[0]
    mats = jnp.asarray(_build_mats())
    return pl.pallas_call(
        _body,
        grid=(b,),
        in_specs=[
            pl.BlockSpec((1, 1, _H, _H), lambda i: (i, 0, 0, 0)),
            pl.BlockSpec(mats.shape, lambda i: (0, 0, 0)),
        ],
        out_specs=pl.BlockSpec((1, 1, 68), lambda i: (i, 0, 0)),
        out_shape=jax.ShapeDtypeStruct((b, 1, 68), jnp.float32),
    )(x, mats).reshape(b, 68)


# speedup vs baseline: 13.4197x; 1.1500x over previous
import functools

import numpy as np
import jax
import jax.numpy as jnp
from jax import lax
from jax.experimental import pallas as pl

_H = 224
_K = 17
_S = 4
_NW = 52
_L = 5
_G_MEAN = 85.384
_G_STD = 53.798
_THRESH = (0.5, _G_MEAN - _G_STD, _G_MEAN, _G_MEAN + _G_STD)

_OFFSETS = [(0, 1), (1, 1), (1, 0), (1, -1),
            (0, 2), (1, 1), (2, 0), (1, -1),
            (0, 4), (3, 3), (4, 0), (3, -3)]
_UNIQUE = []
for _o in _OFFSETS:
    if _o not in _UNIQUE:
        _UNIQUE.append(_o)
_T2U = [_UNIQUE.index(_o) for _o in _OFFSETS]
_NU = len(_UNIQUE)


def _build_mats():
    r = np.arange(_H)[None, :]
    k4 = (_S * np.arange(_NW))[:, None]
    mats = []
    for dr, _dc in _UNIQUE:
        mats.append(((r >= k4) & (r < k4 + _K - dr)).astype(np.float32))
    for _dr, dc in _UNIQUE:
        lo, hi = max(0, -dc), max(0, dc)
        mats.append(((r >= k4 + lo) & (r < k4 + _K - hi)).astype(np.float32))
    mats.append(((r >= k4) & (r < k4 + _K)).astype(np.float32))
    sel = np.zeros((_NW, _H), np.float32)
    sel[np.arange(_NW), _S * np.arange(_NW)] = 1.0
    mats.append(sel)
    return np.stack(mats)


def _dot_hi(a, b):
    return lax.dot_general(a, b, (((1,), (0,)), ((), ())),
                           precision=lax.Precision.HIGHEST,
                           preferred_element_type=jnp.float32)


def _dot_nt_hi(a, b):
    return lax.dot_general(a, b, (((1,), (1,)), ((), ())),
                           precision=lax.Precision.HIGHEST,
                           preferred_element_type=jnp.float32)


def _dot(a, b):
    return lax.dot_general(a, b, (((1,), (0,)), ((), ())),
                           preferred_element_type=jnp.float32)


def _dot_nt(a, b):
    return lax.dot_general(a, b, (((1,), (1,)), ((), ())),
                           preferred_element_type=jnp.float32)


def _shift(m, dr, dc, fill):
    src = m[dr:, max(dc, 0):_H + min(dc, 0)]
    rows = _H - dr
    if dc > 0:
        src = jnp.concatenate(
            [src, jnp.full((rows, dc), fill, m.dtype)], axis=1)
    elif dc < 0:
        src = jnp.concatenate(
            [jnp.full((rows, -dc), fill, m.dtype), src], axis=1)
    if dr > 0:
        src = jnp.concatenate(
            [src, jnp.full((dr, _H), fill, m.dtype)], axis=0)
    return src


def _win_reduce(m, op, axis):
    def sl(a, off, size):
        if axis == 0:
            return a[off:off + size, :]
        return a[:, off:off + size]
    d = m.shape[axis]
    w2 = op(sl(m, 0, d - 1), sl(m, 1, d - 1))
    w4 = op(sl(w2, 0, d - 3), sl(w2, 2, d - 3))
    w8 = op(sl(w4, 0, d - 7), sl(w4, 4, d - 7))
    w16 = op(sl(w8, 0, d - 15), sl(w8, 8, d - 15))
    return op(sl(w16, 0, d - 16), sl(m, 16, d - 16))


def _body(x_ref, mats_ref, out_ref):
    x = x_ref[0, 0]
    m17 = mats_ref[2 * _NU]
    sel = mats_ref[2 * _NU + 1][:, :_H - _K + 1]

    npx = float(_K * _K)
    s1 = _dot_nt_hi(_dot_hi(m17, x), m17)
    s2 = _dot_nt_hi(_dot_hi(m17, x * x), m17)
    mean = s1 / npx
    var = jnp.maximum(s2 / npx - mean * mean, 0.0)
    std = jnp.sqrt(var)

    def box_red(m, op):
        cm = _win_reduce(m, op, axis=1)
        csel = _dot_nt_hi(cm, sel)
        rm = _win_reduce(csel, op, axis=0)
        return _dot_hi(sel, rm)

    wmax = box_red(x, jnp.maximum)
    wmin = box_red(x, jnp.minimum)

    means_n = mean / _G_MEAN
    std_n = std / _G_STD
    mx_n = (wmax - means_n) / _G_STD
    mn_n = (means_n - wmin) / _G_STD

    qlev = jnp.zeros_like(x)
    for t in _THRESH:
        qlev = qlev + (x >= t).astype(jnp.float32)
    onehot = [(qlev == float(a)).astype(jnp.float32) for a in range(_L)]

    hc = [_dot_nt(_dot(m17, onehot[a]), m17) for a in range(1, _L)]
    hs = hc[0] + hc[1] + hc[2] + hc[3]
    hsg = jnp.where(hs == 0.0, 1.0, hs)
    hist = [h / hsg for h in hc]

    obig = jnp.concatenate(
        [onehot[a] for a in range(_L) for _ in range(_L)], axis=0)

    uniq_feats = []
    for u, (dr, dc) in enumerate(_UNIQUE):
        ar = mats_ref[u]
        ac = mats_ref[_NU + u]
        qs = _shift(qlev, dr, dc, -1.0)
        sh = [(qs == float(b)).astype(jnp.float32) for b in range(_L)]
        scat = jnp.concatenate(sh, axis=0)
        sbig = jnp.concatenate([scat] * _L, axis=0)
        pstack = obig * sbig
        tstack = _dot_nt(pstack, ac)
        t3 = tstack.reshape(_L * _L, _H, _NW)
        arb = jnp.broadcast_to(ar, (_L * _L, _NW, _H))
        cnt_st = lax.dot_general(
            arb, t3, (((2,), (1,)), ((0,), (0,))),
            preferred_element_type=jnp.float32)
        inv = 1.0 / float(2 * (_K - dr) * (_K - abs(dc)))
        cnt_t = cnt_st.reshape(_L, _L, _NW, _NW).transpose(
            (1, 0, 2, 3)).reshape(_L * _L, _NW, _NW)
        nst = (cnt_st + cnt_t) * inv
        s_p = jnp.sum(jnp.sum(nst, axis=2), axis=1)
        con = jnp.float32(0.0)
        hom = jnp.float32(0.0)
        for a in range(_L):
            for bb in range(_L):
                d2 = float((a - bb) ** 2)
                spv = s_p[_L * a + bb]
                if d2:
                    con = con + d2 * spv
                hom = hom + (1.0 / (1.0 + d2)) * spv
        ent = -jnp.sum(nst * jnp.log2(nst + 1e-8))
        energy = jnp.sum(jnp.sqrt(jnp.sum(nst * nst, axis=0)))
        n4 = nst.reshape(_L, _L, _NW, _NW)
        ra = jnp.sum(n4, axis=1)
        cb = jnp.sum(n4, axis=0)
        mi = jnp.zeros_like(ra[0])
        mj = jnp.zeros_like(mi)
        for a in range(1, _L):
            mi = mi + float(a) * ra[a]
            mj = mj + float(a) * cb[a]
        vi = jnp.zeros_like(mi)
        vj = jnp.zeros_like(mi)
        for a in range(_L):
            da = float(a) - mi
            vi = vi + da * da * ra[a]
            db = float(a) - mj
            vj = vj + db * db * cb[a]
        cov = jnp.zeros_like(mi)
        for a in range(_L):
            da = float(a) - mi
            for b in range(_L):
                cov = cov + da * (float(b) - mj) * n4[a, b]
        stdi = jnp.sqrt(vi)
        stdj = jnp.sqrt(vj)
        den = stdi * stdj
        corr = jnp.where((stdi < 1e-15) | (stdj < 1e-15), 1.0,
                         cov / jnp.where(den == 0.0, 1.0, den))
        uniq_feats.append((con, hom, energy, jnp.sum(corr), ent))

    inv_w = 1.0 / float(_NW * _NW)
    vals = [jnp.sum(means_n), jnp.sum(std_n), jnp.sum(mx_n), jnp.sum(mn_n)]
    vals += [jnp.sum(h) for h in hist]
    for f in range(5):
        vals += [uniq_feats[_T2U[t]][f] for t in range(len(_OFFSETS))]
    row = jnp.concatenate([(v * inv_w).reshape(1, 1) for v in vals], axis=1)
    out_ref[...] = row[None]


def kernel(x):
    b = x.shape[0]
    mats = jnp.asarray(_build_mats())
    return pl.pallas_call(
        _body,
        grid=(b,),
        in_specs=[
            pl.BlockSpec((1, 1, _H, _H), lambda i: (i, 0, 0, 0)),
            pl.BlockSpec(mats.shape, lambda i: (0, 0, 0)),
        ],
        out_specs=pl.BlockSpec((1, 1, 68), lambda i: (i, 0, 0)),
        out_shape=jax.ShapeDtypeStruct((b, 1, 68), jnp.float32),
    )(x, mats).reshape(b, 68)
